# baseline (speedup 1.0000x reference)
import jax
import jax.numpy as jnp
from jax.experimental import pallas as pl
from jax.experimental.pallas import tpu as pltpu


def _ca_fused_kernel(x_ref, w1t_ref, w2t_ref, o_ref):
    xb = x_ref[...]
    bt = xb.shape[0]
    hw = xb.shape[-1]

    s = jnp.sum(xb, axis=-1, dtype=jnp.float32) * (1.0 / hw)
    m = jnp.max(xb, axis=-1).astype(jnp.float32)

    pools = jnp.concatenate([s, m], axis=0)
    h = jnp.maximum(
        jnp.dot(pools, w1t_ref[...], preferred_element_type=jnp.float32), 0.0)
    attn = jax.nn.sigmoid(
        jnp.dot(h[:bt] + h[bt:], w2t_ref[...],
                preferred_element_type=jnp.float32))

    o_ref[...] = (xb * attn[:, :, None]).astype(o_ref.dtype)


def _pick_bt(B):
    for d in (4, 2, 1):
        if B % d == 0:
            return d
    return 1


def kernel(x, w1, w2):
    B, C, H, W = x.shape
    HW = H * W
    x_flat = x.reshape(B, C, HW)
    w1t = jnp.asarray(w1).T
    w2t = jnp.asarray(w2).T

    Bt = _pick_bt(B)
    out_flat = pl.pallas_call(
        _ca_fused_kernel,
        out_shape=jax.ShapeDtypeStruct((B, C, HW), x.dtype),
        grid=(B // Bt,),
        in_specs=[
            pl.BlockSpec((Bt, C, HW), lambda b: (b, 0, 0)),
            pl.BlockSpec((C, w1t.shape[1]), lambda b: (0, 0)),
            pl.BlockSpec((w2t.shape[0], C), lambda b: (0, 0)),
        ],
        out_specs=pl.BlockSpec((Bt, C, HW), lambda b: (b, 0, 0)),
        compiler_params=pltpu.CompilerParams(
            dimension_semantics=("parallel",),
        ),
    )(x_flat, w1t, w2t)
    return out_flat.reshape(B, C, H, W)
